# Initial kernel scaffold; baseline (speedup 1.0000x reference)
import jax
import jax.numpy as jnp
from jax.experimental import pallas as pl


def kernel(x, mlp1, mlp2, w1_w, w1_b, w2_w, w2_b):
    raise NotImplementedError("write your pallas kernel here")


# speedup vs baseline: 1.0328x; 1.0328x over previous
import jax
import jax.numpy as jnp
from jax.experimental import pallas as pl

_M = 8
_C = 1.25
def kernel(x, mlp1, mlp2, w1_w, w1_b, w2_w, w2_b):
    b, s, d = x.shape
    m = _M
    cap = int(b * s * _C // m)
    x_flat = x.reshape(-1, d)
    hp = jnp.dot(x.astype(jnp.bfloat16), w1_w.T.astype(jnp.bfloat16),
                 precision=jax.lax.Precision.HIGHEST,
                 preferred_element_type=jnp.float32)
    h = jax.nn.gelu(hp + w1_b, approximate=False)
    logits = jnp.dot(h.astype(jnp.bfloat16), w2_w.T.astype(jnp.bfloat16),
                     precision=jax.lax.Precision.HIGHEST,
                     preferred_element_type=jnp.float32) + w2_b
    idx = jnp.argmax(logits, axis=-1).reshape(-1)
    oh = jax.nn.one_hot(idx, m, dtype=jnp.int32)
    counts = jnp.cumsum(oh, axis=0)
    pos = jnp.take_along_axis(counts, idx[:, None], axis=1)[:, 0] - 1
    mask = pos < cap
    dest = jnp.where(mask, idx * cap + pos, m * cap)
    flat_inputs = jnp.zeros((m * cap + 1, d), x.dtype).at[dest].set(x_flat)[: m * cap]
    expert_inputs = flat_inputs.reshape(m, cap, d)
    h1 = jax.nn.gelu(jnp.einsum('mcd,mdf->mcf', expert_inputs, mlp1), approximate=False)
    expert_outputs = jnp.einsum('mcf,mfd->mcd', h1, mlp2)
    flat_out = expert_outputs.reshape(m * cap, d)
    safe = jnp.where(mask, idx * cap + pos, 0)
    gathered = flat_out[safe]
    out = jnp.where(mask[:, None], gathered, x_flat).reshape(b, s, d)
    return out, logits


# speedup vs baseline: 3.0483x; 2.9516x over previous
import functools

import numpy as np

import jax
import jax.numpy as jnp
from jax import lax
from jax.experimental import pallas as pl
from jax.experimental.pallas import tpu as pltpu
from jax.experimental.pallas import tpu_sc as plsc

_NC = 2
_NS = 16
_NW = _NC * _NS

_CAP_FACTOR = 1.25
_TS = 512
_FC = 1024
_ROWCHUNK = 32


def _gelu(v):
    return 0.5 * v * (1.0 + lax.erf(v * np.float32(1.0 / np.sqrt(2.0))))


def _nt(a, b):
    return lax.dot_general(
        a.astype(jnp.bfloat16), b.astype(jnp.bfloat16),
        (((1,), (1,)), ((), ())), preferred_element_type=jnp.float32)


def _nn(a, b):
    return lax.dot_general(
        a.astype(jnp.bfloat16), b.astype(jnp.bfloat16),
        (((1,), (0,)), ((), ())), preferred_element_type=jnp.float32)


def _router_body(m, cap, trash_row, xpass_base,
                 x_ref, w1_ref, b1_ref, w2_ref, b2_ref,
                 logits_ref, dest_ref, g_ref, run_ref):
    i = pl.program_id(0)

    @pl.when(i == 0)
    def _():
        run_ref[...] = jnp.zeros_like(run_ref)

    h = _gelu(_nt(x_ref[...], w1_ref[...]) + b1_ref[...])
    lg = _nt(h, w2_ref[...]) + b2_ref[...]
    logits_ref[...] = lg

    mx = jnp.max(lg, axis=1, keepdims=True)
    lane = lax.broadcasted_iota(jnp.int32, lg.shape, 1)
    idxv = jnp.min(jnp.where(lg == mx, lane, m), axis=1, keepdims=True)

    oh = (lane == idxv).astype(jnp.float32)
    r = lax.broadcasted_iota(jnp.int32, (_TS, _TS), 0)
    c = lax.broadcasted_iota(jnp.int32, (_TS, _TS), 1)
    tri = (r >= c).astype(jnp.float32)
    counts = _nn(tri, oh)
    run = run_ref[...]
    pos = jnp.sum(oh * (counts + run), axis=1, keepdims=True) - 1.0
    run_ref[...] = run + counts[_TS - 1:_TS, :]

    pos = pos.astype(jnp.int32)
    maskv = pos < cap
    destv = idxv * cap + pos
    dest_ref[...] = jnp.where(maskv, destv, trash_row)
    tok = i * _TS + lax.broadcasted_iota(jnp.int32, (_TS, 1), 0)
    g_ref[...] = jnp.where(maskv, destv, xpass_base + tok)


def _router(x_flat, w1_w, w1_b, w2_w, w2_b, m, cap, trash_row, xpass_base):
    t, d = x_flat.shape
    dm = w1_w.shape[0]
    grid = (t // _TS,)
    body = functools.partial(_router_body, m, cap, trash_row, xpass_base)
    return pl.pallas_call(
        body,
        grid=grid,
        in_specs=[
            pl.BlockSpec((_TS, d), lambda i: (i, 0)),
            pl.BlockSpec((dm, d), lambda i: (0, 0)),
            pl.BlockSpec((1, dm), lambda i: (0, 0)),
            pl.BlockSpec((m, dm), lambda i: (0, 0)),
            pl.BlockSpec((1, m), lambda i: (0, 0)),
        ],
        out_specs=[
            pl.BlockSpec((_TS, m), lambda i: (i, 0)),
            pl.BlockSpec((_TS, 1), lambda i: (i, 0)),
            pl.BlockSpec((_TS, 1), lambda i: (i, 0)),
        ],
        out_shape=[
            jax.ShapeDtypeStruct((t, m), jnp.float32),
            jax.ShapeDtypeStruct((t, 1), jnp.int32),
            jax.ShapeDtypeStruct((t, 1), jnp.int32),
        ],
        scratch_shapes=[pltpu.VMEM((1, m), jnp.float32)],
    )(x_flat, w1_w, w1_b, w2_w, w2_b)


def _dispatch(x_flat, dest3, n_rows, xpass_base):
    t, d = x_flat.shape
    per_w = t // _NW
    nchunk = per_w // _ROWCHUNK
    mesh = plsc.VectorSubcoreMesh(core_axis_name="c", subcore_axis_name="s")

    @functools.partial(
        pl.kernel, mesh=mesh,
        out_type=jax.ShapeDtypeStruct((n_rows, d), jnp.float32),
        scratch_types=[
            pltpu.VMEM((nchunk, _ROWCHUNK), jnp.int32),
            pltpu.VMEM((_ROWCHUNK, d), jnp.float32),
            pltpu.SemaphoreType.DMA,
        ],
    )
    def k(x_hbm, dest_hbm, t1_hbm, dest_v, rows_v, sem):
        wid = lax.axis_index("s") * _NC + lax.axis_index("c")
        base = wid * per_w
        pltpu.sync_copy(dest_hbm.at[wid], dest_v)
        for j in range(nchunk):
            r0 = base + j * _ROWCHUNK
            pltpu.sync_copy(x_hbm.at[pl.ds(r0, _ROWCHUNK)], rows_v)
            pltpu.sync_copy(rows_v, t1_hbm.at[pl.ds(xpass_base + r0, _ROWCHUNK)])
            pltpu.async_copy(rows_v, t1_hbm.at[dest_v.at[j]], sem).wait()

    return k(x_flat, dest3)


def _experts(t1, mlp1, mlp2, m, cap):
    n_rows, d = t1.shape
    dm = mlp1.shape[2]
    nf = dm // _FC

    def body(t1_ref, w1_ref, w2_ref, out_ref, acc_ref):
        f = pl.program_id(1)
        h = _gelu(_nn(t1_ref[...], w1_ref[0]))
        p = _nn(h, w2_ref[0])

        @pl.when(f == 0)
        def _():
            acc_ref[...] = p

        @pl.when(f > 0)
        def _():
            acc_ref[...] += p

        @pl.when(f == nf - 1)
        def _():
            out_ref[...] = acc_ref[...]

    return pl.pallas_call(
        body,
        grid=(m, nf),
        in_specs=[
            pl.BlockSpec((cap, d), lambda e, f: (e, 0)),
            pl.BlockSpec((1, d, _FC), lambda e, f: (e, 0, f)),
            pl.BlockSpec((1, _FC, d), lambda e, f: (e, f, 0)),
        ],
        out_specs=pl.BlockSpec((cap, d), lambda e, f: (e, 0)),
        out_shape=jax.ShapeDtypeStruct((n_rows, d), jnp.float32),
        scratch_shapes=[pltpu.VMEM((cap, d), jnp.float32)],
        input_output_aliases={0: 0},
    )(t1, mlp1, mlp2)


def _combine(t2, g3, t, d):
    per_w = t // _NW
    nchunk = per_w // _ROWCHUNK
    mesh = plsc.VectorSubcoreMesh(core_axis_name="c", subcore_axis_name="s")

    @functools.partial(
        pl.kernel, mesh=mesh,
        out_type=jax.ShapeDtypeStruct((t, d), jnp.float32),
        scratch_types=[
            pltpu.VMEM((nchunk, _ROWCHUNK), jnp.int32),
            pltpu.VMEM((_ROWCHUNK, d), jnp.float32),
            pltpu.SemaphoreType.DMA,
        ],
    )
    def k(t2_hbm, g_hbm, out_hbm, g_v, rows_v, sem):
        wid = lax.axis_index("s") * _NC + lax.axis_index("c")
        base = wid * per_w
        pltpu.sync_copy(g_hbm.at[wid], g_v)
        for j in range(nchunk):
            pltpu.async_copy(t2_hbm.at[g_v.at[j]], rows_v, sem).wait()
            pltpu.sync_copy(rows_v, out_hbm.at[pl.ds(base + j * _ROWCHUNK, _ROWCHUNK)])

    return k(t2, g3)


def kernel(x, mlp1, mlp2, w1_w, w1_b, w2_w, w2_b):
    b, s, d = x.shape
    t = b * s
    m = mlp1.shape[0]
    cap = int(t * _CAP_FACTOR // m)
    trash_row = m * cap
    xpass_base = m * cap + 8
    n_rows = xpass_base + t

    x_flat = x.reshape(t, d)
    logits, dest, g = _router(
        x_flat, w1_w, w1_b.reshape(1, -1), w2_w, w2_b.reshape(1, -1),
        m, cap, trash_row, xpass_base)

    dest3 = dest.reshape(_NW, t // _NW // _ROWCHUNK, _ROWCHUNK)
    g3 = g.reshape(_NW, t // _NW // _ROWCHUNK, _ROWCHUNK)

    t1 = _dispatch(x_flat, dest3, n_rows, xpass_base)
    t2 = _experts(t1, mlp1, mlp2, m, cap)
    out_flat = _combine(t2, g3, t, d)
    return out_flat.reshape(b, s, d), logits.reshape(b, s, m)


# speedup vs baseline: 3.2556x; 1.0680x over previous
import functools

import numpy as np

import jax
import jax.numpy as jnp
from jax import lax
from jax.experimental import pallas as pl
from jax.experimental.pallas import tpu as pltpu
from jax.experimental.pallas import tpu_sc as plsc

_NC = 2
_NS = 16
_NW = _NC * _NS

_CAP_FACTOR = 1.25
_TS = 512
_FC = 2048
_ROWCHUNK = 16
_NBUF = 6


def _gelu(v):
    return 0.5 * v * (1.0 + lax.erf(v * np.float32(1.0 / np.sqrt(2.0))))


def _nt(a, b):
    return lax.dot_general(
        a.astype(jnp.bfloat16), b.astype(jnp.bfloat16),
        (((1,), (1,)), ((), ())), preferred_element_type=jnp.float32)


def _nn(a, b):
    return lax.dot_general(
        a.astype(jnp.bfloat16), b.astype(jnp.bfloat16),
        (((1,), (0,)), ((), ())), preferred_element_type=jnp.float32)


def _router_body(m, cap, trash_row, xpass_base,
                 x_ref, w1_ref, b1_ref, w2_ref, b2_ref,
                 logits_ref, dest_ref, g_ref, run_ref, w1b_ref):
    i = pl.program_id(0)

    @pl.when(i == 0)
    def _():
        run_ref[...] = jnp.zeros_like(run_ref)
        w1b_ref[...] = w1_ref[...].astype(jnp.bfloat16)

    hp = lax.dot_general(
        x_ref[...].astype(jnp.bfloat16), w1b_ref[...],
        (((1,), (1,)), ((), ())), preferred_element_type=jnp.float32)
    h = _gelu(hp + b1_ref[...])
    lg = _nt(h, w2_ref[...]) + b2_ref[...]
    logits_ref[...] = lg

    mx = jnp.max(lg, axis=1, keepdims=True)
    lane = lax.broadcasted_iota(jnp.int32, lg.shape, 1)
    idxv = jnp.min(jnp.where(lg == mx, lane, m), axis=1, keepdims=True)

    oh = (lane == idxv).astype(jnp.float32)
    r = lax.broadcasted_iota(jnp.int32, (_TS, _TS), 0)
    c = lax.broadcasted_iota(jnp.int32, (_TS, _TS), 1)
    tri = (r >= c).astype(jnp.float32)
    counts = _nn(tri, oh)
    run = run_ref[...]
    pos = jnp.sum(oh * (counts + run), axis=1, keepdims=True) - 1.0
    run_ref[...] = run + counts[_TS - 1:_TS, :]

    pos = pos.astype(jnp.int32)
    maskv = pos < cap
    destv = idxv * cap + pos
    dest_ref[...] = jnp.where(maskv, destv, trash_row)
    tok = i * _TS + lax.broadcasted_iota(jnp.int32, (_TS, 1), 0)
    g_ref[...] = jnp.where(maskv, destv, xpass_base + tok)


def _router(x_flat, w1_w, w1_b, w2_w, w2_b, m, cap, trash_row, xpass_base):
    t, d = x_flat.shape
    dm = w1_w.shape[0]
    grid = (t // _TS,)
    body = functools.partial(_router_body, m, cap, trash_row, xpass_base)
    return pl.pallas_call(
        body,
        grid=grid,
        in_specs=[
            pl.BlockSpec((_TS, d), lambda i: (i, 0)),
            pl.BlockSpec((dm, d), lambda i: (0, 0)),
            pl.BlockSpec((1, dm), lambda i: (0, 0)),
            pl.BlockSpec((m, dm), lambda i: (0, 0)),
            pl.BlockSpec((1, m), lambda i: (0, 0)),
        ],
        out_specs=[
            pl.BlockSpec((_TS, m), lambda i: (i, 0)),
            pl.BlockSpec((_TS, 1), lambda i: (i, 0)),
            pl.BlockSpec((_TS, 1), lambda i: (i, 0)),
        ],
        out_shape=[
            jax.ShapeDtypeStruct((t, m), jnp.float32),
            jax.ShapeDtypeStruct((t, 1), jnp.int32),
            jax.ShapeDtypeStruct((t, 1), jnp.int32),
        ],
        scratch_shapes=[pltpu.VMEM((1, m), jnp.float32),
                        pltpu.VMEM((dm, d), jnp.bfloat16)],
    )(x_flat, w1_w, w1_b, w2_w, w2_b)


def _dispatch(x_flat, dest3, n_rows, xpass_base):
    t, d = x_flat.shape
    per_w = t // _NW
    nchunk = per_w // _ROWCHUNK
    mesh = plsc.VectorSubcoreMesh(core_axis_name="c", subcore_axis_name="s")

    nbuf = min(_NBUF, nchunk)

    @functools.partial(
        pl.kernel, mesh=mesh,
        out_type=jax.ShapeDtypeStruct((n_rows, d), jnp.float32),
        scratch_types=[
            pltpu.VMEM((nchunk, _ROWCHUNK), jnp.int32),
            pltpu.VMEM((nbuf, _ROWCHUNK, d), jnp.float32),
            pltpu.SemaphoreType.DMA((nbuf,)),
            pltpu.SemaphoreType.DMA((nbuf,)),
        ],
    )
    def k(x_hbm, dest_hbm, t1_hbm, dest_v, bufs, lsem, wsem):
        wid = lax.axis_index("s") * _NC + lax.axis_index("c")
        base = wid * per_w
        pltpu.sync_copy(dest_hbm.at[wid], dest_v)
        ld, pd, sd = [None] * nchunk, [None] * nchunk, [None] * nchunk
        for j in range(nbuf):
            ld[j] = pltpu.async_copy(
                x_hbm.at[pl.ds(base + j * _ROWCHUNK, _ROWCHUNK)],
                bufs.at[j], lsem.at[j])
        for j in range(nchunk):
            b = j % nbuf
            if j >= nbuf:
                pd[j - nbuf].wait()
                sd[j - nbuf].wait()
                ld[j] = pltpu.async_copy(
                    x_hbm.at[pl.ds(base + j * _ROWCHUNK, _ROWCHUNK)],
                    bufs.at[b], lsem.at[b])
            ld[j].wait()
            pd[j] = pltpu.async_copy(
                bufs.at[b],
                t1_hbm.at[pl.ds(xpass_base + base + j * _ROWCHUNK, _ROWCHUNK)],
                wsem.at[b])
            sd[j] = pltpu.async_copy(bufs.at[b], t1_hbm.at[dest_v.at[j]], wsem.at[b])
        for j in range(nchunk - nbuf, nchunk):
            pd[j].wait()
            sd[j].wait()

    return k(x_flat, dest3)


def _experts(t1, mlp1, mlp2, m, cap):
    n_rows, d = t1.shape
    dm = mlp1.shape[2]
    nf = dm // _FC

    def body(t1_ref, w1_ref, w2_ref, out_ref):
        f = pl.program_id(1)
        h = _gelu(_nn(t1_ref[...], w1_ref[0]))
        p = _nn(h, w2_ref[0])

        @pl.when(f == 0)
        def _():
            out_ref[...] = p

        @pl.when(f > 0)
        def _():
            out_ref[...] += p

    return pl.pallas_call(
        body,
        grid=(m, nf),
        in_specs=[
            pl.BlockSpec((cap, d), lambda e, f: (e, 0)),
            pl.BlockSpec((1, d, _FC), lambda e, f: (e, 0, f)),
            pl.BlockSpec((1, _FC, d), lambda e, f: (e, f, 0)),
        ],
        out_specs=pl.BlockSpec((cap, d), lambda e, f: (e, 0)),
        out_shape=jax.ShapeDtypeStruct((n_rows, d), jnp.float32),
        input_output_aliases={0: 0},
    )(t1, mlp1, mlp2)


def _combine(t2, g3, t, d):
    per_w = t // _NW
    nchunk = per_w // _ROWCHUNK
    mesh = plsc.VectorSubcoreMesh(core_axis_name="c", subcore_axis_name="s")

    nbuf = min(_NBUF, nchunk)

    @functools.partial(
        pl.kernel, mesh=mesh,
        out_type=jax.ShapeDtypeStruct((t, d), jnp.float32),
        scratch_types=[
            pltpu.VMEM((nchunk, _ROWCHUNK), jnp.int32),
            pltpu.VMEM((nbuf, _ROWCHUNK, d), jnp.float32),
            pltpu.SemaphoreType.DMA((nbuf,)),
            pltpu.SemaphoreType.DMA((nbuf,)),
        ],
    )
    def k(t2_hbm, g_hbm, out_hbm, g_v, bufs, gsem, wsem):
        wid = lax.axis_index("s") * _NC + lax.axis_index("c")
        base = wid * per_w
        pltpu.sync_copy(g_hbm.at[wid], g_v)
        gd, wd = [None] * nchunk, [None] * nchunk
        for j in range(nbuf):
            gd[j] = pltpu.async_copy(t2_hbm.at[g_v.at[j]], bufs.at[j], gsem.at[j])
        for j in range(nchunk):
            b = j % nbuf
            if j >= nbuf:
                wd[j - nbuf].wait()
                gd[j] = pltpu.async_copy(t2_hbm.at[g_v.at[j]], bufs.at[b], gsem.at[b])
            gd[j].wait()
            wd[j] = pltpu.async_copy(
                bufs.at[b], out_hbm.at[pl.ds(base + j * _ROWCHUNK, _ROWCHUNK)],
                wsem.at[b])
        for j in range(nchunk - nbuf, nchunk):
            wd[j].wait()

    return k(t2, g3)


def kernel(x, mlp1, mlp2, w1_w, w1_b, w2_w, w2_b):
    b, s, d = x.shape
    t = b * s
    m = mlp1.shape[0]
    cap = int(t * _CAP_FACTOR // m)
    trash_row = m * cap
    xpass_base = m * cap + 8
    n_rows = xpass_base + t

    x_flat = x.reshape(t, d)
    logits, dest, g = _router(
        x_flat, w1_w, w1_b.reshape(1, -1), w2_w, w2_b.reshape(1, -1),
        m, cap, trash_row, xpass_base)

    dest3 = dest.reshape(_NW, t // _NW // _ROWCHUNK, _ROWCHUNK)
    g3 = g.reshape(_NW, t // _NW // _ROWCHUNK, _ROWCHUNK)

    t1 = _dispatch(x_flat, dest3, n_rows, xpass_base)
    t2 = _experts(t1, mlp1, mlp2, m, cap)
    out_flat = _combine(t2, g3, t, d)
    return out_flat.reshape(b, s, d), logits.reshape(b, s, m)
